# Initial kernel scaffold; baseline (speedup 1.0000x reference)
import jax
import jax.numpy as jnp
from jax.experimental import pallas as pl


def kernel(X, TmT, R_diag, Q_diag, x_tar, grid, psi_tab):
    raise NotImplementedError("write your pallas kernel here")


# speedup vs baseline: 121.4807x; 121.4807x over previous
import functools

import jax
import jax.numpy as jnp
from jax import lax
from jax.experimental import pallas as pl
from jax.experimental.pallas import tpu as pltpu
from jax.experimental.pallas import tpu_sc as plsc

_T = 1.0
_NG = 2001
_NGP = 2008
_D = 8
_NC, _NS, _L = 2, 16, 16
_NW = _NC * _NS
_CH = 1024


def _dyn_gather(v, idx):
    dn = lax.GatherDimensionNumbers(
        offset_dims=(), collapsed_slice_dims=(0,), start_index_map=(0,))
    return lax.gather(v, idx[:, None], dn, (1,),
                      mode=lax.GatherScatterMode.PROMISE_IN_BOUNDS)


def _sc_body(rows_per_w, x_hbm, t_hbm, p_hbm, out_hbm, p_v, x_v, o_v, t_v):
    wid = lax.axis_index("s") * _NC + lax.axis_index("c")
    pltpu.sync_copy(p_hbm, p_v)

    lane = lax.iota(jnp.int32, _L)
    dlane = lane & 7
    half = lane >> 3

    def chunk_body(c, carry):
        row0 = wid * rows_per_w + c * _CH
        pltpu.sync_copy(x_hbm.at[pl.ds(row0 * _D, _CH * _D)], x_v)
        pltpu.sync_copy(t_hbm.at[pl.ds(row0, _CH)], t_v)

        def blk(k, carry2):
            t16 = t_v[pl.ds(k * _L, _L)]
            f = 2000.0 - t16 * 2000.0
            ji = jnp.clip(f.astype(jnp.int32), 0, _NG - 2)
            w16 = f - ji.astype(jnp.float32)
            for p in range(8):
                pair = half + 2 * p
                jv = _dyn_gather(ji, pair)
                wv = _dyn_gather(w16, pair)
                gb = jv * 16 + dlane
                a0 = plsc.load_gather(p_v, [gb])
                a1 = plsc.load_gather(p_v, [gb + 16])
                b0 = plsc.load_gather(p_v, [gb + 8])
                b1 = plsc.load_gather(p_v, [gb + 24])
                off = k * (_L * 8) + p * _L
                xv = x_v[pl.ds(off, _L)]
                u = (a0 + wv * (a1 - a0)) * xv + (b0 + wv * (b1 - b0))
                o_v[pl.ds(off, _L)] = u
            return carry2

        lax.fori_loop(0, _CH // _L, blk, 0)
        pltpu.sync_copy(o_v, out_hbm.at[pl.ds(row0 * _D, _CH * _D)])
        return carry

    lax.fori_loop(0, rows_per_w // _CH, chunk_body, 0)


def kernel(X, TmT, R_diag, Q_diag, x_tar, grid, psi_tab):
    B = X.shape[0]
    sqrtQR = jnp.sqrt(Q_diag / R_diag)
    S = sqrtQR * jnp.tanh(sqrtQR * (_T - grid)[:, None])
    price = 10.0 + 2.0 * jnp.sin(2.0 * jnp.pi * grid)
    A = -S
    C = S * x_tar + (price[:, None] - psi_tab) / R_diag
    P = jnp.concatenate([A, C], axis=1)
    P = jnp.concatenate([P, jnp.zeros((_NGP - _NG, 16), jnp.float32)], axis=0)
    P = P.reshape(-1)

    rows_per_w = B // _NW
    mesh = plsc.VectorSubcoreMesh(core_axis_name="c", subcore_axis_name="s",
                                  num_cores=_NC, num_subcores=_NS)
    call = pl.kernel(
        functools.partial(_sc_body, rows_per_w),
        out_type=jax.ShapeDtypeStruct((B * _D,), jnp.float32),
        mesh=mesh,
        compiler_params=pltpu.CompilerParams(needs_layout_passes=False),
        scratch_types=[
            pltpu.VMEM((_NGP * 16,), jnp.float32),
            pltpu.VMEM((_CH * _D,), jnp.float32),
            pltpu.VMEM((_CH * _D,), jnp.float32),
            pltpu.VMEM((_CH,), jnp.float32),
        ],
    )
    out = call(X.reshape(-1), TmT.reshape(-1), P)
    return out.reshape(B, _D)
